# baseline (speedup 1.0000x reference)
import functools

import jax
import jax.numpy as jnp
from jax import lax
from jax.experimental import pallas as pl
from jax.experimental.pallas import tpu as pltpu
from jax.experimental.pallas import tpu_sc as plsc

B = 1024
S = 200
DIM = 64

_info = plsc.get_sparse_core_info()
NC, NS, L = _info.num_cores, _info.num_subcores, _info.num_lanes
NW = NC * NS

TOTAL = B * S
TOK_PER_W = TOTAL // NW
CHUNK = 40
ROWS_PER_CHUNK = S // CHUNK
NCHUNKS = TOK_PER_W // CHUNK
VPR = DIM // 16


def _make_kernel():
  mesh = plsc.VectorSubcoreMesh(core_axis_name="c", subcore_axis_name="s")

  @functools.partial(
      pl.kernel,
      mesh=mesh,
      compiler_params=pltpu.CompilerParams(use_tc_tiling_on_sc=False),
      out_type=jax.ShapeDtypeStruct((TOTAL, DIM), jnp.float32),
      scratch_types=[
          pltpu.VMEM((S, DIM), jnp.float32),
          pltpu.VMEM((CHUNK,), jnp.int32),
          pltpu.VMEM((CHUNK, DIM), jnp.float32),
          pltpu.SemaphoreType.DMA,
      ],
  )
  def k(ids_hbm, table_hbm, pos_hbm, out_hbm, pos_v, idx_v, rows_v, sem):
    wid = lax.axis_index("s") * NC + lax.axis_index("c")
    pltpu.sync_copy(pos_hbm, pos_v)
    base0 = wid * TOK_PER_W

    def chunk_body(c, carry):
      base = base0 + c * CHUNK
      prow0 = lax.rem(c, ROWS_PER_CHUNK) * CHUNK
      pltpu.sync_copy(ids_hbm.at[pl.ds(base, CHUNK)], idx_v)
      pltpu.async_copy(table_hbm.at[idx_v], rows_v, sem).wait()

      def row_body(r, carry2):
        pr = prow0 + r
        for kk in range(VPR):
          sl = pl.ds(kk * 16, 16)
          rows_v[r, sl] = rows_v[r, sl] + pos_v[pr, sl]
        return carry2

      lax.fori_loop(0, CHUNK, row_body, 0, unroll=4)
      pltpu.sync_copy(rows_v, out_hbm.at[pl.ds(base, CHUNK)])
      return carry

    lax.fori_loop(0, NCHUNKS, chunk_body, 0)

  return k


_kernel = _make_kernel()


def kernel(input_ids, token_table, position_embedding):
  Bq, Sq = input_ids.shape
  ids_flat = input_ids.reshape(-1).astype(jnp.int32)
  pos = position_embedding[0, :Sq, :]
  out = _kernel(ids_flat, token_table, pos)
  return out.reshape(Bq, Sq, DIM)


# speedup vs baseline: 1.2059x; 1.2059x over previous
import functools

import jax
import jax.numpy as jnp
from jax import lax
from jax.experimental import pallas as pl
from jax.experimental.pallas import tpu as pltpu
from jax.experimental.pallas import tpu_sc as plsc

B = 1024
S = 200
DIM = 64

_info = plsc.get_sparse_core_info()
NC, NS, L = _info.num_cores, _info.num_subcores, _info.num_lanes
NW = NC * NS

TOTAL = B * S
TOK_PER_W = TOTAL // NW
CHUNK = 100
PHASES = S // CHUNK
NCHUNKS = TOK_PER_W // CHUNK
CPW = NCHUNKS
VPR = DIM // 16


def _make_kernel():
  mesh = plsc.VectorSubcoreMesh(core_axis_name="c", subcore_axis_name="s")

  @functools.partial(
      pl.kernel,
      mesh=mesh,
      compiler_params=pltpu.CompilerParams(use_tc_tiling_on_sc=False),
      out_type=jax.ShapeDtypeStruct((TOTAL, DIM), jnp.float32),
      scratch_types=[
          pltpu.VMEM((S, DIM), jnp.float32),
          pltpu.VMEM((CPW, CHUNK), jnp.int32),
          pltpu.VMEM((CHUNK, DIM), jnp.float32),
          pltpu.VMEM((CHUNK, DIM), jnp.float32),
          pltpu.SemaphoreType.DMA,
          pltpu.SemaphoreType.DMA,
      ],
  )
  def k(ids_hbm, table_hbm, pos_hbm, out_hbm,
        pos_v, idx_v, rows0, rows1, sem0, sem1):
    wid = lax.axis_index("s") * NC + lax.axis_index("c")
    pltpu.sync_copy(pos_hbm, pos_v)
    pltpu.sync_copy(ids_hbm.at[pl.ds(wid * CPW, CPW)], idx_v)
    base0 = wid * TOK_PER_W

    bufs = (rows0, rows1)
    sems = (sem0, sem1)

    pltpu.async_copy(table_hbm.at[idx_v.at[0]], rows0, sem0)

    def wait_gather(buf, sem):
      pltpu.make_async_copy(table_hbm.at[pl.ds(0, CHUNK)], buf, sem).wait()

    def step(c, b):
      cur, csem = bufs[b], sems[b]
      nxt, nsem = bufs[1 - b], sems[1 - b]

      @pl.when(c + 1 < NCHUNKS)
      def _():
        pltpu.async_copy(table_hbm.at[idx_v.at[c + 1]], nxt, nsem)

      wait_gather(cur, csem)
      prow0 = lax.rem(c, PHASES) * CHUNK

      def row_body(r, carry2):
        pr = prow0 + r
        for kk in range(VPR):
          sl = pl.ds(kk * 16, 16)
          cur[r, sl] = cur[r, sl] + pos_v[pr, sl]
        return carry2

      lax.fori_loop(0, CHUNK, row_body, 0, unroll=4)
      pltpu.sync_copy(cur, out_hbm.at[pl.ds(base0 + c * CHUNK, CHUNK)])

    def pair_body(j, carry):
      for b in range(2):
        step(2 * j + b, b)
      return carry

    lax.fori_loop(0, NCHUNKS // 2, pair_body, 0)

  return k


_kernel = _make_kernel()


def kernel(input_ids, token_table, position_embedding):
  Bq, Sq = input_ids.shape
  ids2d = input_ids.reshape(-1).astype(jnp.int32).reshape(-1, CHUNK)
  pos = position_embedding[0, :Sq, :]
  out = _kernel(ids2d, token_table, pos)
  return out.reshape(Bq, Sq, DIM)


# speedup vs baseline: 1.3569x; 1.1252x over previous
import functools

import jax
import jax.numpy as jnp
from jax import lax
from jax.experimental import pallas as pl
from jax.experimental.pallas import tpu as pltpu
from jax.experimental.pallas import tpu_sc as plsc

B = 1024
S = 200
DIM = 64

_info = plsc.get_sparse_core_info()
NC, NS, L = _info.num_cores, _info.num_subcores, _info.num_lanes
NW = NC * NS

TOTAL = B * S
TOK_PER_W = TOTAL // NW
CHUNK = 100
PHASES = S // CHUNK
NCHUNKS = TOK_PER_W // CHUNK
CPW = NCHUNKS
VPR = DIM // 16
RING = 4
LEAD = 2


def _make_kernel():
  mesh = plsc.VectorSubcoreMesh(core_axis_name="c", subcore_axis_name="s")

  rows_scratch = [pltpu.VMEM((CHUNK, DIM), jnp.float32) for _ in range(RING)]
  sem_scratch = [pltpu.SemaphoreType.DMA for _ in range(2 * RING)]

  @functools.partial(
      pl.kernel,
      mesh=mesh,
      compiler_params=pltpu.CompilerParams(use_tc_tiling_on_sc=False),
      out_type=jax.ShapeDtypeStruct((TOTAL, DIM), jnp.float32),
      scratch_types=[
          pltpu.VMEM((S, DIM), jnp.float32),
          pltpu.VMEM((CPW, CHUNK), jnp.int32),
      ] + rows_scratch + sem_scratch,
  )
  def k(ids_hbm, table_hbm, pos_hbm, out_hbm, pos_v, idx_v, *rest):
    bufs = rest[:RING]
    gsems = rest[RING:2 * RING]
    ssems = rest[2 * RING:]
    wid = lax.axis_index("s") * NC + lax.axis_index("c")
    pltpu.sync_copy(pos_hbm, pos_v)
    pltpu.sync_copy(ids_hbm.at[pl.ds(wid * CPW, CPW)], idx_v)
    base0 = wid * TOK_PER_W

    def issue_gather(c, b):
      pltpu.async_copy(table_hbm.at[idx_v.at[c]], bufs[b], gsems[b])

    def wait_gather(b):
      pltpu.make_async_copy(
          table_hbm.at[pl.ds(0, CHUNK)], bufs[b], gsems[b]).wait()

    def wait_store(b):
      pltpu.make_async_copy(
          bufs[b], out_hbm.at[pl.ds(0, CHUNK)], ssems[b]).wait()

    for c in range(LEAD):
      issue_gather(c, c % RING)

    def step(c, b):
      @pl.when(c + LEAD < NCHUNKS)
      def _():
        bp = (b + LEAD) % RING

        @pl.when(c >= RING - LEAD)
        def _():
          wait_store(bp)

        issue_gather(c + LEAD, bp)

      wait_gather(b)
      cur = bufs[b]
      prow0 = lax.rem(c, PHASES) * CHUNK

      def row_body(r, carry2):
        pr = prow0 + r
        for kk in range(VPR):
          sl = pl.ds(kk * 16, 16)
          plsc.addupdate(cur.at[r, sl], pos_v[pr, sl])
        return carry2

      lax.fori_loop(0, CHUNK, row_body, 0, unroll=8)
      pltpu.async_copy(cur, out_hbm.at[pl.ds(base0 + c * CHUNK, CHUNK)],
                       ssems[b])

    def ring_body(j, carry):
      for b in range(RING):
        step(j * RING + b, b)
      return carry

    lax.fori_loop(0, NCHUNKS // RING, ring_body, 0)

    for b in range(RING):
      wait_store(b)

  return k


_kernel = _make_kernel()


def kernel(input_ids, token_table, position_embedding):
  Bq, Sq = input_ids.shape
  ids2d = input_ids.reshape(-1).astype(jnp.int32).reshape(-1, CHUNK)
  pos = position_embedding[0, :Sq, :]
  out = _kernel(ids2d, token_table, pos)
  return out.reshape(Bq, Sq, DIM)
